# baseline (speedup 1.0000x reference)
import jax
import jax.numpy as jnp
from jax.experimental import pallas as pl
from jax.experimental.pallas import tpu as pltpu


def _copy_body(w_ref, o_ref, sem):
    pltpu.make_async_copy(w_ref, o_ref, sem).start()
    pltpu.make_async_copy(w_ref, o_ref, sem).wait()


def kernel(x, W_pos):
    seq_len, d_model = W_pos.shape
    return pl.pallas_call(
        _copy_body,
        in_specs=[pl.BlockSpec(memory_space=pltpu.MemorySpace.HBM)],
        out_specs=pl.BlockSpec(memory_space=pltpu.MemorySpace.HBM),
        out_shape=jax.ShapeDtypeStruct((seq_len, d_model), W_pos.dtype),
        scratch_shapes=[pltpu.SemaphoreType.DMA],
    )(W_pos)
